# baseline (speedup 1.0000x reference)
import jax
import jax.numpy as jnp
from jax.experimental import pallas as pl

_CHUNK = 2000
_ZBLK = 2000


def _body(zc_ref, table_ref, out1_ref, out2_ref):
    i = pl.program_id(0)
    out2_ref[...] = jnp.zeros_like(out2_ref)

    @pl.when(i < 5)
    def _():
        idx = zc_ref[...]
        tv = table_ref[...]
        v = tv.shape[0]
        onehot = (idx == jax.lax.broadcasted_iota(
            jnp.int32, (idx.shape[0], v), 1))
        out1_ref[...] = jax.lax.dot_general(
            onehot.astype(jnp.float32), tv,
            dimension_numbers=(((1,), (0,)), ((), ())),
            preferred_element_type=jnp.float32)


def kernel(z, graph, edges_dist, orientation, table):
    del orientation
    zi = z.astype(jnp.int32)
    B = zi.shape[0]
    V, D = table.shape
    E = graph.shape[0]
    zc = zi.reshape(B, 1)
    n_blk = E // _ZBLK
    node_scalars, node_vectors = pl.pallas_call(
        _body,
        grid=(n_blk,),
        in_specs=[
            pl.BlockSpec((_CHUNK, 1), lambda i: (jnp.minimum(i, 4), 0)),
            pl.BlockSpec((V, D), lambda i: (0, 0)),
        ],
        out_specs=[
            pl.BlockSpec((_CHUNK, D), lambda i: (jnp.minimum(i, 4), 0)),
            pl.BlockSpec((_ZBLK, 3, D), lambda i: (i, 0, 0)),
        ],
        out_shape=[
            jax.ShapeDtypeStruct((B, D), jnp.float32),
            jax.ShapeDtypeStruct((E, 3, D), edges_dist.dtype),
        ],
    )(zc, table)
    return (node_scalars, node_vectors)
